# baseline (speedup 1.0000x reference)
import jax
import jax.numpy as jnp
from jax.experimental import pallas as pl
from jax.experimental.pallas import tpu as pltpu

TOPK_K = 32
ROW_BLK = 256
TIE_CHUNK = 512
P2_I = 256
P2_J = 1024


def _pow2(e):
    return jax.lax.bitcast_convert_type((e + 127) << 23, jnp.float32)


def _sparsify_body(z_all_ref, z_blk_ref, m1_ref, m2_ref, a_scr):
    i = pl.program_id(1)
    za = z_all_ref[0].astype(jnp.bfloat16)
    zb = z_blk_ref[0].astype(jnp.bfloat16)
    w = jax.lax.dot_general(
        za, zb, (((1,), (1,)), ((), ())),
        preferred_element_type=jnp.float32)
    a = jax.nn.sigmoid(w)
    n = a.shape[0]
    ci = jax.lax.broadcasted_iota(jnp.int32, a.shape, 0)
    ri = jax.lax.broadcasted_iota(jnp.int32, a.shape, 1) + i * ROW_BLK
    a = jnp.where(ci == ri, 0.0, a)
    a_scr[...] = a

    lo0 = jnp.zeros((1, ROW_BLK), jnp.int32)
    hi0 = jnp.full((1, ROW_BLK), 0x3F800000, jnp.int32)

    def search(_, lohi):
        lo, hi = lohi
        mid = (lo + hi + 1) >> 1
        thr = jax.lax.bitcast_convert_type(mid, jnp.float32)
        cnt = jnp.sum((a_scr[...] >= thr).astype(jnp.int32), axis=0,
                      keepdims=True)
        ge = cnt >= TOPK_K
        return jnp.where(ge, mid, lo), jnp.where(ge, hi, mid - 1)

    lo, _ = jax.lax.fori_loop(0, 31, search, (lo0, hi0))
    v32 = jax.lax.bitcast_convert_type(lo, jnp.float32)

    av = a_scr[...]
    gt = av > v32
    eq = av == v32
    cnt_gt = jnp.sum(gt.astype(jnp.int32), axis=0, keepdims=True)
    r = (TOPK_K - cnt_gt).astype(jnp.float32)

    t0 = jax.lax.broadcasted_iota(jnp.int32, (TIE_CHUNK, TIE_CHUNK), 0)
    t1 = jax.lax.broadcasted_iota(jnp.int32, (TIE_CHUNK, TIE_CHUNK), 1)
    tril = (t1 < t0).astype(jnp.bfloat16)
    eqb = eq.astype(jnp.bfloat16)
    keeps = []
    carry = jnp.zeros((1, ROW_BLK), jnp.float32)
    for c in range(n // TIE_CHUNK):
        eqc = eqb[c * TIE_CHUNK:(c + 1) * TIE_CHUNK, :]
        excl = jax.lax.dot_general(
            tril, eqc, (((1,), (0,)), ((), ())),
            preferred_element_type=jnp.float32) + carry
        sl = slice(c * TIE_CHUNK, (c + 1) * TIE_CHUNK)
        keeps.append(jnp.logical_or(gt[sl], jnp.logical_and(eq[sl],
                                                            excl < r)))
        carry = carry + jnp.sum(eqc.astype(jnp.float32), axis=0,
                                keepdims=True)
    keep = jnp.concatenate(keeps, axis=0).astype(jnp.bfloat16)

    gi = jax.lax.broadcasted_iota(jnp.int32, (n, n // 8), 0)
    gj = jax.lax.broadcasted_iota(jnp.int32, (n, n // 8), 1)
    bsub = jnp.where(gi >> 3 == gj, _pow2(gi & 7), 0.0).astype(jnp.bfloat16)
    m1 = jax.lax.dot_general(
        keep, bsub, (((0,), (0,)), ((), ())),
        preferred_element_type=jnp.float32)
    m1_ref[0] = m1.astype(jnp.bfloat16)
    hi2 = jax.lax.broadcasted_iota(jnp.int32, (ROW_BLK, ROW_BLK // 8), 0)
    hj2 = jax.lax.broadcasted_iota(jnp.int32, (ROW_BLK, ROW_BLK // 8), 1)
    b2 = jnp.where(hi2 >> 3 == hj2, _pow2(hi2 & 7), 0.0).astype(jnp.bfloat16)
    m2 = jax.lax.dot_general(
        b2, keep, (((0,), (1,)), ((), ())),
        preferred_element_type=jnp.float32)
    m2_ref[0] = m2.astype(jnp.bfloat16)


def _bit_of(bytes_f32, p):
    t = jnp.floor(bytes_f32 * _pow2(-p))
    return t - 2.0 * jnp.floor(t * 0.5)


def _output_body(z_i_ref, z_j_ref, m1_ref, m2_ref, o_ref):
    zi = z_i_ref[0].astype(jnp.bfloat16)
    zj = z_j_ref[0].astype(jnp.bfloat16)
    s = jax.lax.dot_general(
        zi, zj, (((1,), (1,)), ((), ())),
        preferred_element_type=jnp.float32)
    a = jax.nn.sigmoid(s)

    cols = jax.lax.broadcasted_iota(jnp.int32, (P2_I, P2_J), 1)

    ei = jax.lax.broadcasted_iota(jnp.int32, (P2_J // 8, P2_J), 0)
    ej = jax.lax.broadcasted_iota(jnp.int32, (P2_J // 8, P2_J), 1)
    e1 = (ej >> 3 == ei).astype(jnp.bfloat16)
    b1 = jax.lax.dot_general(
        m1_ref[0].astype(jnp.bfloat16), e1, (((1,), (0,)), ((), ())),
        preferred_element_type=jnp.float32)
    mask = _bit_of(b1, cols & 7)
    b2 = jax.lax.dot_general(
        m2_ref[0].astype(jnp.bfloat16), e1, (((0,), (0,)), ((), ())),
        preferred_element_type=jnp.float32)
    mask_t = _bit_of(b2, cols & 7)

    o_ref[0] = jnp.where(mask + mask_t > 0.0, a, 0.0)


def _build(z, interpret=False):
    b, n, d = z.shape
    nrb = n // ROW_BLK

    m1, m2 = pl.pallas_call(
        _sparsify_body,
        grid=(b, nrb),
        in_specs=[
            pl.BlockSpec((1, n, d), lambda bi, i: (bi, 0, 0)),
            pl.BlockSpec((1, ROW_BLK, d), lambda bi, i: (bi, i, 0)),
        ],
        out_specs=[
            pl.BlockSpec((1, ROW_BLK, n // 8), lambda bi, i: (bi, i, 0)),
            pl.BlockSpec((1, ROW_BLK // 8, n), lambda bi, i: (bi, i, 0)),
        ],
        out_shape=[
            jax.ShapeDtypeStruct((b, n, n // 8), jnp.bfloat16),
            jax.ShapeDtypeStruct((b, n // 8, n), jnp.bfloat16),
        ],
        scratch_shapes=[
            pltpu.VMEM((n, ROW_BLK), jnp.float32),
        ],
        interpret=interpret,
    )(z, z)

    out = pl.pallas_call(
        _output_body,
        grid=(b, n // P2_I, n // P2_J),
        in_specs=[
            pl.BlockSpec((1, P2_I, d), lambda bi, i, j: (bi, i, 0)),
            pl.BlockSpec((1, P2_J, d), lambda bi, i, j: (bi, j, 0)),
            pl.BlockSpec((1, P2_I, P2_J // 8), lambda bi, i, j: (bi, i, j)),
            pl.BlockSpec((1, P2_J // 8, P2_I), lambda bi, i, j: (bi, j, i)),
        ],
        out_specs=pl.BlockSpec((1, P2_I, P2_J), lambda bi, i, j: (bi, i, j)),
        out_shape=jax.ShapeDtypeStruct((b, n, n), jnp.float32),
        interpret=interpret,
    )(z, z, m1, m2)
    return out


@jax.jit
def kernel(z):
    return _build(z)


# speedup vs baseline: 1.6057x; 1.6057x over previous
import jax
import jax.numpy as jnp
from jax.experimental import pallas as pl
from jax.experimental.pallas import tpu as pltpu

TOPK_K = 32
ROW_BLK = 256
TIE_CHUNK = 512
P2_I = 256
P2_J = 1024


def _pow2(e):
    return jax.lax.bitcast_convert_type((e + 127) << 23, jnp.float32)


def _sparsify_body(z_all_ref, z_blk_ref, m1_ref, m2_ref, a_scr):
    i = pl.program_id(1)
    za = z_all_ref[0].astype(jnp.bfloat16)
    zb = z_blk_ref[0].astype(jnp.bfloat16)
    w = jax.lax.dot_general(
        za, zb, (((1,), (1,)), ((), ())),
        preferred_element_type=jnp.float32)
    a = jax.nn.sigmoid(w)
    n = a.shape[0]
    ci = jax.lax.broadcasted_iota(jnp.int32, a.shape, 0)
    ri = jax.lax.broadcasted_iota(jnp.int32, a.shape, 1) + i * ROW_BLK
    a = jnp.where(ci == ri, 0.0, a)
    a_scr[...] = a

    x = a
    for half in (1024, 512, 256, 128, 64):
        x = jnp.maximum(x[:half], x[half:])
    cmax = x
    hi0 = jax.lax.bitcast_convert_type(
        jnp.max(cmax, axis=0, keepdims=True), jnp.int32)

    def seed_search(_, lohi):
        lo, hi = lohi
        mid = (lo + hi + 1) >> 1
        thr = jax.lax.bitcast_convert_type(mid, jnp.float32)
        cnt = jnp.sum((cmax >= thr).astype(jnp.int32), axis=0,
                      keepdims=True)
        ge = cnt >= TOPK_K
        return jnp.where(ge, mid, lo), jnp.where(ge, hi, mid - 1)

    lo0, _ = jax.lax.fori_loop(
        0, 31, seed_search, (jnp.zeros((1, ROW_BLK), jnp.int32), hi0))

    def not_done(lohi):
        lo, hi = lohi
        return jnp.any(lo < hi)

    def search(lohi):
        lo, hi = lohi
        mid = (lo + hi + 1) >> 1
        thr = jax.lax.bitcast_convert_type(mid, jnp.float32)
        cnt = jnp.sum((a_scr[...] >= thr).astype(jnp.int32), axis=0,
                      keepdims=True)
        ge = cnt >= TOPK_K
        return jnp.where(ge, mid, lo), jnp.where(ge, hi, mid - 1)

    lo, _ = jax.lax.while_loop(not_done, search, (lo0, hi0))
    v32 = jax.lax.bitcast_convert_type(lo, jnp.float32)

    av = a_scr[...]
    gt = av > v32
    eq = av == v32
    cnt_gt = jnp.sum(gt.astype(jnp.int32), axis=0, keepdims=True)
    r = (TOPK_K - cnt_gt).astype(jnp.float32)

    t0 = jax.lax.broadcasted_iota(jnp.int32, (TIE_CHUNK, TIE_CHUNK), 0)
    t1 = jax.lax.broadcasted_iota(jnp.int32, (TIE_CHUNK, TIE_CHUNK), 1)
    tril = (t1 < t0).astype(jnp.bfloat16)
    eqb = eq.astype(jnp.bfloat16)
    keeps = []
    carry = jnp.zeros((1, ROW_BLK), jnp.float32)
    for c in range(n // TIE_CHUNK):
        eqc = eqb[c * TIE_CHUNK:(c + 1) * TIE_CHUNK, :]
        excl = jax.lax.dot_general(
            tril, eqc, (((1,), (0,)), ((), ())),
            preferred_element_type=jnp.float32) + carry
        sl = slice(c * TIE_CHUNK, (c + 1) * TIE_CHUNK)
        keeps.append(jnp.logical_or(gt[sl], jnp.logical_and(eq[sl],
                                                            excl < r)))
        carry = carry + jnp.sum(eqc.astype(jnp.float32), axis=0,
                                keepdims=True)
    keep = jnp.concatenate(keeps, axis=0).astype(jnp.bfloat16)

    gi = jax.lax.broadcasted_iota(jnp.int32, (n, n // 8), 0)
    gj = jax.lax.broadcasted_iota(jnp.int32, (n, n // 8), 1)
    bsub = jnp.where(gi >> 3 == gj, _pow2(gi & 7), 0.0).astype(jnp.bfloat16)
    m1 = jax.lax.dot_general(
        keep, bsub, (((0,), (0,)), ((), ())),
        preferred_element_type=jnp.float32)
    m1_ref[0] = m1.astype(jnp.bfloat16)
    hi2 = jax.lax.broadcasted_iota(jnp.int32, (ROW_BLK, ROW_BLK // 8), 0)
    hj2 = jax.lax.broadcasted_iota(jnp.int32, (ROW_BLK, ROW_BLK // 8), 1)
    b2 = jnp.where(hi2 >> 3 == hj2, _pow2(hi2 & 7), 0.0).astype(jnp.bfloat16)
    m2 = jax.lax.dot_general(
        b2, keep, (((0,), (1,)), ((), ())),
        preferred_element_type=jnp.float32)
    m2_ref[0] = m2.astype(jnp.bfloat16)


def _bit_of(bytes_f32, p):
    t = jnp.floor(bytes_f32 * _pow2(-p))
    return t - 2.0 * jnp.floor(t * 0.5)


def _output_body(z_i_ref, z_j_ref, m1_ref, m2_ref, o_ref):
    zi = z_i_ref[0].astype(jnp.bfloat16)
    zj = z_j_ref[0].astype(jnp.bfloat16)
    s = jax.lax.dot_general(
        zi, zj, (((1,), (1,)), ((), ())),
        preferred_element_type=jnp.float32)
    a = jax.nn.sigmoid(s)

    cols = jax.lax.broadcasted_iota(jnp.int32, (P2_I, P2_J), 1)

    ei = jax.lax.broadcasted_iota(jnp.int32, (P2_J // 8, P2_J), 0)
    ej = jax.lax.broadcasted_iota(jnp.int32, (P2_J // 8, P2_J), 1)
    e1 = (ej >> 3 == ei).astype(jnp.bfloat16)
    b1 = jax.lax.dot_general(
        m1_ref[0].astype(jnp.bfloat16), e1, (((1,), (0,)), ((), ())),
        preferred_element_type=jnp.float32)
    mask = _bit_of(b1, cols & 7)
    b2 = jax.lax.dot_general(
        m2_ref[0].astype(jnp.bfloat16), e1, (((0,), (0,)), ((), ())),
        preferred_element_type=jnp.float32)
    mask_t = _bit_of(b2, cols & 7)

    o_ref[0] = jnp.where(mask + mask_t > 0.0, a, 0.0)


def _build(z, interpret=False):
    b, n, d = z.shape
    nrb = n // ROW_BLK

    m1, m2 = pl.pallas_call(
        _sparsify_body,
        grid=(b, nrb),
        in_specs=[
            pl.BlockSpec((1, n, d), lambda bi, i: (bi, 0, 0)),
            pl.BlockSpec((1, ROW_BLK, d), lambda bi, i: (bi, i, 0)),
        ],
        out_specs=[
            pl.BlockSpec((1, ROW_BLK, n // 8), lambda bi, i: (bi, i, 0)),
            pl.BlockSpec((1, ROW_BLK // 8, n), lambda bi, i: (bi, i, 0)),
        ],
        out_shape=[
            jax.ShapeDtypeStruct((b, n, n // 8), jnp.bfloat16),
            jax.ShapeDtypeStruct((b, n // 8, n), jnp.bfloat16),
        ],
        scratch_shapes=[
            pltpu.VMEM((n, ROW_BLK), jnp.float32),
        ],
        interpret=interpret,
    )(z, z)

    out = pl.pallas_call(
        _output_body,
        grid=(b, n // P2_I, n // P2_J),
        in_specs=[
            pl.BlockSpec((1, P2_I, d), lambda bi, i, j: (bi, i, 0)),
            pl.BlockSpec((1, P2_J, d), lambda bi, i, j: (bi, j, 0)),
            pl.BlockSpec((1, P2_I, P2_J // 8), lambda bi, i, j: (bi, i, j)),
            pl.BlockSpec((1, P2_J // 8, P2_I), lambda bi, i, j: (bi, j, i)),
        ],
        out_specs=pl.BlockSpec((1, P2_I, P2_J), lambda bi, i, j: (bi, i, j)),
        out_shape=jax.ShapeDtypeStruct((b, n, n), jnp.float32),
        interpret=interpret,
    )(z, z, m1, m2)
    return out


@jax.jit
def kernel(z):
    return _build(z)


# speedup vs baseline: 1.8496x; 1.1519x over previous
import jax
import jax.numpy as jnp
from jax.experimental import pallas as pl
from jax.experimental.pallas import tpu as pltpu

TOPK_K = 32
ROW_BLK = 512
TIE_CHUNK = 512
P2_I = 256
P2_J = 1024


def _pow2(e):
    return jax.lax.bitcast_convert_type((e + 127) << 23, jnp.float32)


def _sparsify_body(z_all_ref, z_blk_ref, m1_ref, m2_ref, a_scr):
    i = pl.program_id(1)
    za = z_all_ref[0].astype(jnp.bfloat16)
    zb = z_blk_ref[0].astype(jnp.bfloat16)
    w = jax.lax.dot_general(
        za, zb, (((1,), (1,)), ((), ())),
        preferred_element_type=jnp.float32)
    a = jax.nn.sigmoid(w)
    n = a.shape[0]
    ci = jax.lax.broadcasted_iota(jnp.int32, a.shape, 0)
    ri = jax.lax.broadcasted_iota(jnp.int32, a.shape, 1) + i * ROW_BLK
    a = jnp.where(ci == ri, 0.0, a)
    a_scr[...] = a

    x = a
    for half in (1024, 512, 256, 128, 64):
        x = jnp.maximum(x[:half], x[half:])
    cmax = x
    hi0 = jax.lax.bitcast_convert_type(
        jnp.max(cmax, axis=0, keepdims=True), jnp.int32)

    def seed_search(_, lohi):
        lo, hi = lohi
        mid = (lo + hi + 1) >> 1
        thr = jax.lax.bitcast_convert_type(mid, jnp.float32)
        cnt = jnp.sum((cmax >= thr).astype(jnp.int32), axis=0,
                      keepdims=True)
        ge = cnt >= TOPK_K
        return jnp.where(ge, mid, lo), jnp.where(ge, hi, mid - 1)

    lo0, _ = jax.lax.fori_loop(
        0, 31, seed_search, (jnp.zeros((1, ROW_BLK), jnp.int32), hi0))

    def not_done(lohi):
        lo, hi = lohi
        return jnp.any(lo < hi)

    def search(lohi):
        lo, hi = lohi
        mid = (lo + hi + 1) >> 1
        thr = jax.lax.bitcast_convert_type(mid, jnp.float32)
        cnt = jnp.sum((a_scr[...] >= thr).astype(jnp.int32), axis=0,
                      keepdims=True)
        ge = cnt >= TOPK_K
        return jnp.where(ge, mid, lo), jnp.where(ge, hi, mid - 1)

    lo, _ = jax.lax.while_loop(not_done, search, (lo0, hi0))
    v32 = jax.lax.bitcast_convert_type(lo, jnp.float32)

    av = a_scr[...]
    gt = av > v32
    eq = av == v32
    cnt_gt = jnp.sum(gt.astype(jnp.int32), axis=0, keepdims=True)
    r = (TOPK_K - cnt_gt).astype(jnp.float32)

    t0 = jax.lax.broadcasted_iota(jnp.int32, (TIE_CHUNK, TIE_CHUNK), 0)
    t1 = jax.lax.broadcasted_iota(jnp.int32, (TIE_CHUNK, TIE_CHUNK), 1)
    tril = (t1 < t0).astype(jnp.bfloat16)
    eqb = eq.astype(jnp.bfloat16)
    keeps = []
    carry = jnp.zeros((1, ROW_BLK), jnp.float32)
    for c in range(n // TIE_CHUNK):
        eqc = eqb[c * TIE_CHUNK:(c + 1) * TIE_CHUNK, :]
        excl = jax.lax.dot_general(
            tril, eqc, (((1,), (0,)), ((), ())),
            preferred_element_type=jnp.float32) + carry
        sl = slice(c * TIE_CHUNK, (c + 1) * TIE_CHUNK)
        keeps.append(jnp.logical_or(gt[sl], jnp.logical_and(eq[sl],
                                                            excl < r)))
        carry = carry + jnp.sum(eqc.astype(jnp.float32), axis=0,
                                keepdims=True)
    keep = jnp.concatenate(keeps, axis=0).astype(jnp.bfloat16)

    gi = jax.lax.broadcasted_iota(jnp.int32, (n, n // 8), 0)
    gj = jax.lax.broadcasted_iota(jnp.int32, (n, n // 8), 1)
    bsub = jnp.where(gi >> 3 == gj, _pow2(gi & 7), 0.0).astype(jnp.bfloat16)
    m1 = jax.lax.dot_general(
        keep, bsub, (((0,), (0,)), ((), ())),
        preferred_element_type=jnp.float32)
    m1_ref[0] = m1.astype(jnp.bfloat16)
    hi2 = jax.lax.broadcasted_iota(jnp.int32, (ROW_BLK, ROW_BLK // 8), 0)
    hj2 = jax.lax.broadcasted_iota(jnp.int32, (ROW_BLK, ROW_BLK // 8), 1)
    b2 = jnp.where(hi2 >> 3 == hj2, _pow2(hi2 & 7), 0.0).astype(jnp.bfloat16)
    m2 = jax.lax.dot_general(
        b2, keep, (((0,), (1,)), ((), ())),
        preferred_element_type=jnp.float32)
    m2_ref[0] = m2.astype(jnp.bfloat16)


def _bit_of(bytes_f32, p):
    t = jnp.floor(bytes_f32 * _pow2(-p))
    return t - 2.0 * jnp.floor(t * 0.5)


def _output_body(z_i_ref, z_j_ref, m1_ref, m2_ref, o_ref):
    zi = z_i_ref[0].astype(jnp.bfloat16)
    zj = z_j_ref[0].astype(jnp.bfloat16)
    s = jax.lax.dot_general(
        zi, zj, (((1,), (1,)), ((), ())),
        preferred_element_type=jnp.float32)
    a = jax.nn.sigmoid(s)

    cols = jax.lax.broadcasted_iota(jnp.int32, (P2_I, P2_J), 1)

    ei = jax.lax.broadcasted_iota(jnp.int32, (P2_J // 8, P2_J), 0)
    ej = jax.lax.broadcasted_iota(jnp.int32, (P2_J // 8, P2_J), 1)
    e1 = (ej >> 3 == ei).astype(jnp.bfloat16)
    b1 = jax.lax.dot_general(
        m1_ref[0].astype(jnp.bfloat16), e1, (((1,), (0,)), ((), ())),
        preferred_element_type=jnp.float32)
    mask = _bit_of(b1, cols & 7)
    b2 = jax.lax.dot_general(
        m2_ref[0].astype(jnp.bfloat16), e1, (((0,), (0,)), ((), ())),
        preferred_element_type=jnp.float32)
    mask_t = _bit_of(b2, cols & 7)

    o_ref[0] = jnp.where(mask + mask_t > 0.0, a, 0.0)


def _build(z, interpret=False):
    b, n, d = z.shape
    nrb = n // ROW_BLK

    m1, m2 = pl.pallas_call(
        _sparsify_body,
        grid=(b, nrb),
        in_specs=[
            pl.BlockSpec((1, n, d), lambda bi, i: (bi, 0, 0)),
            pl.BlockSpec((1, ROW_BLK, d), lambda bi, i: (bi, i, 0)),
        ],
        out_specs=[
            pl.BlockSpec((1, ROW_BLK, n // 8), lambda bi, i: (bi, i, 0)),
            pl.BlockSpec((1, ROW_BLK // 8, n), lambda bi, i: (bi, i, 0)),
        ],
        out_shape=[
            jax.ShapeDtypeStruct((b, n, n // 8), jnp.bfloat16),
            jax.ShapeDtypeStruct((b, n // 8, n), jnp.bfloat16),
        ],
        scratch_shapes=[
            pltpu.VMEM((n, ROW_BLK), jnp.float32),
        ],
        interpret=interpret,
    )(z, z)

    out = pl.pallas_call(
        _output_body,
        grid=(b, n // P2_I, n // P2_J),
        in_specs=[
            pl.BlockSpec((1, P2_I, d), lambda bi, i, j: (bi, i, 0)),
            pl.BlockSpec((1, P2_J, d), lambda bi, i, j: (bi, j, 0)),
            pl.BlockSpec((1, P2_I, P2_J // 8), lambda bi, i, j: (bi, i, j)),
            pl.BlockSpec((1, P2_J // 8, P2_I), lambda bi, i, j: (bi, j, i)),
        ],
        out_specs=pl.BlockSpec((1, P2_I, P2_J), lambda bi, i, j: (bi, i, j)),
        out_shape=jax.ShapeDtypeStruct((b, n, n), jnp.float32),
        interpret=interpret,
    )(z, z, m1, m2)
    return out


@jax.jit
def kernel(z):
    return _build(z)


# speedup vs baseline: 1.9196x; 1.0379x over previous
import jax
import jax.numpy as jnp
from jax.experimental import pallas as pl
from jax.experimental.pallas import tpu as pltpu

TOPK_K = 32
ROW_BLK = 1024
TIE_CHUNK = 512
P2_I = 256
P2_J = 1024


def _pow2(e):
    return jax.lax.bitcast_convert_type((e + 127) << 23, jnp.float32)


def _sparsify_body(z_all_ref, z_blk_ref, m1_ref, m2_ref, a_scr):
    i = pl.program_id(1)
    za = z_all_ref[0].astype(jnp.bfloat16)
    zb = z_blk_ref[0].astype(jnp.bfloat16)
    w = jax.lax.dot_general(
        za, zb, (((1,), (1,)), ((), ())),
        preferred_element_type=jnp.float32)
    a = jax.nn.sigmoid(w)
    n = a.shape[0]
    ci = jax.lax.broadcasted_iota(jnp.int32, a.shape, 0)
    ri = jax.lax.broadcasted_iota(jnp.int32, a.shape, 1) + i * ROW_BLK
    a = jnp.where(ci == ri, 0.0, a)
    a_scr[...] = a

    x = a
    for half in (1024, 512, 256, 128, 64):
        x = jnp.maximum(x[:half], x[half:])
    cmax = x
    hi0 = jax.lax.bitcast_convert_type(
        jnp.max(cmax, axis=0, keepdims=True), jnp.int32)

    def seed_search(_, lohi):
        lo, hi = lohi
        mid = (lo + hi + 1) >> 1
        thr = jax.lax.bitcast_convert_type(mid, jnp.float32)
        cnt = jnp.sum((cmax >= thr).astype(jnp.int32), axis=0,
                      keepdims=True)
        ge = cnt >= TOPK_K
        return jnp.where(ge, mid, lo), jnp.where(ge, hi, mid - 1)

    lo0, _ = jax.lax.fori_loop(
        0, 31, seed_search, (jnp.zeros((1, ROW_BLK), jnp.int32), hi0))

    def not_done(lohi):
        lo, hi = lohi
        return jnp.any(lo < hi)

    def search(lohi):
        lo, hi = lohi
        mid = (lo + hi + 1) >> 1
        thr = jax.lax.bitcast_convert_type(mid, jnp.float32)
        cnt = jnp.sum((a_scr[...] >= thr).astype(jnp.int32), axis=0,
                      keepdims=True)
        ge = cnt >= TOPK_K
        return jnp.where(ge, mid, lo), jnp.where(ge, hi, mid - 1)

    lo, _ = jax.lax.while_loop(not_done, search, (lo0, hi0))
    v32 = jax.lax.bitcast_convert_type(lo, jnp.float32)

    av = a_scr[...]
    gt = av > v32
    eq = av == v32
    cnt_gt = jnp.sum(gt.astype(jnp.int32), axis=0, keepdims=True)
    r = (TOPK_K - cnt_gt).astype(jnp.float32)

    t0 = jax.lax.broadcasted_iota(jnp.int32, (TIE_CHUNK, TIE_CHUNK), 0)
    t1 = jax.lax.broadcasted_iota(jnp.int32, (TIE_CHUNK, TIE_CHUNK), 1)
    tril = (t1 < t0).astype(jnp.bfloat16)
    eqb = eq.astype(jnp.bfloat16)
    keeps = []
    carry = jnp.zeros((1, ROW_BLK), jnp.float32)
    for c in range(n // TIE_CHUNK):
        eqc = eqb[c * TIE_CHUNK:(c + 1) * TIE_CHUNK, :]
        excl = jax.lax.dot_general(
            tril, eqc, (((1,), (0,)), ((), ())),
            preferred_element_type=jnp.float32) + carry
        sl = slice(c * TIE_CHUNK, (c + 1) * TIE_CHUNK)
        keeps.append(jnp.logical_or(gt[sl], jnp.logical_and(eq[sl],
                                                            excl < r)))
        carry = carry + jnp.sum(eqc.astype(jnp.float32), axis=0,
                                keepdims=True)
    keep = jnp.concatenate(keeps, axis=0).astype(jnp.bfloat16)

    gi = jax.lax.broadcasted_iota(jnp.int32, (n, n // 8), 0)
    gj = jax.lax.broadcasted_iota(jnp.int32, (n, n // 8), 1)
    bsub = jnp.where(gi >> 3 == gj, _pow2(gi & 7), 0.0).astype(jnp.bfloat16)
    m1 = jax.lax.dot_general(
        keep, bsub, (((0,), (0,)), ((), ())),
        preferred_element_type=jnp.float32)
    m1_ref[0] = m1.astype(jnp.bfloat16)
    hi2 = jax.lax.broadcasted_iota(jnp.int32, (ROW_BLK, ROW_BLK // 8), 0)
    hj2 = jax.lax.broadcasted_iota(jnp.int32, (ROW_BLK, ROW_BLK // 8), 1)
    b2 = jnp.where(hi2 >> 3 == hj2, _pow2(hi2 & 7), 0.0).astype(jnp.bfloat16)
    m2 = jax.lax.dot_general(
        b2, keep, (((0,), (1,)), ((), ())),
        preferred_element_type=jnp.float32)
    m2_ref[0] = m2.astype(jnp.bfloat16)


def _bit_of(bytes_f32, p):
    t = jnp.floor(bytes_f32 * _pow2(-p))
    return t - 2.0 * jnp.floor(t * 0.5)


def _output_body(z_i_ref, z_j_ref, m1_ref, m2_ref, o_ref):
    zi = z_i_ref[0].astype(jnp.bfloat16)
    zj = z_j_ref[0].astype(jnp.bfloat16)
    s = jax.lax.dot_general(
        zi, zj, (((1,), (1,)), ((), ())),
        preferred_element_type=jnp.float32)
    a = jax.nn.sigmoid(s)

    cols = jax.lax.broadcasted_iota(jnp.int32, (P2_I, P2_J), 1)

    ei = jax.lax.broadcasted_iota(jnp.int32, (P2_J // 8, P2_J), 0)
    ej = jax.lax.broadcasted_iota(jnp.int32, (P2_J // 8, P2_J), 1)
    e1 = (ej >> 3 == ei).astype(jnp.bfloat16)
    b1 = jax.lax.dot_general(
        m1_ref[0].astype(jnp.bfloat16), e1, (((1,), (0,)), ((), ())),
        preferred_element_type=jnp.float32)
    mask = _bit_of(b1, cols & 7)
    b2 = jax.lax.dot_general(
        m2_ref[0].astype(jnp.bfloat16), e1, (((0,), (0,)), ((), ())),
        preferred_element_type=jnp.float32)
    mask_t = _bit_of(b2, cols & 7)

    o_ref[0] = jnp.where(mask + mask_t > 0.0, a, 0.0)


def _build(z, interpret=False):
    b, n, d = z.shape
    nrb = n // ROW_BLK

    m1, m2 = pl.pallas_call(
        _sparsify_body,
        grid=(b, nrb),
        in_specs=[
            pl.BlockSpec((1, n, d), lambda bi, i: (bi, 0, 0)),
            pl.BlockSpec((1, ROW_BLK, d), lambda bi, i: (bi, i, 0)),
        ],
        out_specs=[
            pl.BlockSpec((1, ROW_BLK, n // 8), lambda bi, i: (bi, i, 0)),
            pl.BlockSpec((1, ROW_BLK // 8, n), lambda bi, i: (bi, i, 0)),
        ],
        out_shape=[
            jax.ShapeDtypeStruct((b, n, n // 8), jnp.bfloat16),
            jax.ShapeDtypeStruct((b, n // 8, n), jnp.bfloat16),
        ],
        scratch_shapes=[
            pltpu.VMEM((n, ROW_BLK), jnp.float32),
        ],
        interpret=interpret,
    )(z, z)

    out = pl.pallas_call(
        _output_body,
        grid=(b, n // P2_I, n // P2_J),
        in_specs=[
            pl.BlockSpec((1, P2_I, d), lambda bi, i, j: (bi, i, 0)),
            pl.BlockSpec((1, P2_J, d), lambda bi, i, j: (bi, j, 0)),
            pl.BlockSpec((1, P2_I, P2_J // 8), lambda bi, i, j: (bi, i, j)),
            pl.BlockSpec((1, P2_J // 8, P2_I), lambda bi, i, j: (bi, j, i)),
        ],
        out_specs=pl.BlockSpec((1, P2_I, P2_J), lambda bi, i, j: (bi, i, j)),
        out_shape=jax.ShapeDtypeStruct((b, n, n), jnp.float32),
        interpret=interpret,
    )(z, z, m1, m2)
    return out


@jax.jit
def kernel(z):
    return _build(z)


# speedup vs baseline: 2.1145x; 1.1015x over previous
import jax
import jax.numpy as jnp
from jax.experimental import pallas as pl
from jax.experimental.pallas import tpu as pltpu

TOPK_K = 32
ROW_BLK = 1024
TIE_CHUNK = 512
P2_I = 512
P2_J = 1024


def _pow2(e):
    return jax.lax.bitcast_convert_type((e + 127) << 23, jnp.float32)


def _sparsify_body(z_all_ref, z_blk_ref, m1_ref, m2_ref, a_scr):
    i = pl.program_id(1)
    za = z_all_ref[0].astype(jnp.bfloat16)
    zb = z_blk_ref[0].astype(jnp.bfloat16)
    w = jax.lax.dot_general(
        za, zb, (((1,), (1,)), ((), ())),
        preferred_element_type=jnp.float32)
    a = jax.nn.sigmoid(w)
    n = a.shape[0]
    ci = jax.lax.broadcasted_iota(jnp.int32, a.shape, 0)
    ri = jax.lax.broadcasted_iota(jnp.int32, a.shape, 1) + i * ROW_BLK
    a = jnp.where(ci == ri, 0.0, a)
    a_scr[...] = a

    x = a
    for half in (1024, 512, 256, 128, 64):
        x = jnp.maximum(x[:half], x[half:])
    cmax = x
    hi0 = jax.lax.bitcast_convert_type(
        jnp.max(cmax, axis=0, keepdims=True), jnp.int32)

    def seed_search(_, lohi):
        lo, hi = lohi
        mid = (lo + hi + 1) >> 1
        thr = jax.lax.bitcast_convert_type(mid, jnp.float32)
        cnt = jnp.sum((cmax >= thr).astype(jnp.int32), axis=0,
                      keepdims=True)
        ge = cnt >= TOPK_K
        return jnp.where(ge, mid, lo), jnp.where(ge, hi, mid - 1)

    lo0, _ = jax.lax.fori_loop(
        0, 31, seed_search, (jnp.zeros((1, ROW_BLK), jnp.int32), hi0))

    def not_done(lohi):
        lo, hi = lohi
        return jnp.any(lo < hi)

    def search(lohi):
        lo, hi = lohi
        mid = (lo + hi + 1) >> 1
        thr = jax.lax.bitcast_convert_type(mid, jnp.float32)
        cnt = jnp.sum((a_scr[...] >= thr).astype(jnp.int32), axis=0,
                      keepdims=True)
        ge = cnt >= TOPK_K
        return jnp.where(ge, mid, lo), jnp.where(ge, hi, mid - 1)

    lo, _ = jax.lax.while_loop(not_done, search, (lo0, hi0))
    v32 = jax.lax.bitcast_convert_type(lo, jnp.float32)

    av = a_scr[...]
    gt = av > v32
    eq = av == v32
    cnt_gt = jnp.sum(gt.astype(jnp.int32), axis=0, keepdims=True)
    r = (TOPK_K - cnt_gt).astype(jnp.float32)

    t0 = jax.lax.broadcasted_iota(jnp.int32, (TIE_CHUNK, TIE_CHUNK), 0)
    t1 = jax.lax.broadcasted_iota(jnp.int32, (TIE_CHUNK, TIE_CHUNK), 1)
    tril = (t1 < t0).astype(jnp.bfloat16)
    eqb = eq.astype(jnp.bfloat16)
    keeps = []
    carry = jnp.zeros((1, ROW_BLK), jnp.float32)
    for c in range(n // TIE_CHUNK):
        eqc = eqb[c * TIE_CHUNK:(c + 1) * TIE_CHUNK, :]
        excl = jax.lax.dot_general(
            tril, eqc, (((1,), (0,)), ((), ())),
            preferred_element_type=jnp.float32) + carry
        sl = slice(c * TIE_CHUNK, (c + 1) * TIE_CHUNK)
        keeps.append(jnp.logical_or(gt[sl], jnp.logical_and(eq[sl],
                                                            excl < r)))
        carry = carry + jnp.sum(eqc.astype(jnp.float32), axis=0,
                                keepdims=True)
    keep = jnp.concatenate(keeps, axis=0).astype(jnp.bfloat16)

    gi = jax.lax.broadcasted_iota(jnp.int32, (n, n // 8), 0)
    gj = jax.lax.broadcasted_iota(jnp.int32, (n, n // 8), 1)
    bsub = jnp.where(gi >> 3 == gj, _pow2(gi & 7), 0.0).astype(jnp.bfloat16)
    m1 = jax.lax.dot_general(
        keep, bsub, (((0,), (0,)), ((), ())),
        preferred_element_type=jnp.float32)
    m1_ref[0] = m1.astype(jnp.bfloat16)
    hi2 = jax.lax.broadcasted_iota(jnp.int32, (ROW_BLK, ROW_BLK // 8), 0)
    hj2 = jax.lax.broadcasted_iota(jnp.int32, (ROW_BLK, ROW_BLK // 8), 1)
    b2 = jnp.where(hi2 >> 3 == hj2, _pow2(hi2 & 7), 0.0).astype(jnp.bfloat16)
    m2 = jax.lax.dot_general(
        b2, keep, (((0,), (1,)), ((), ())),
        preferred_element_type=jnp.float32)
    m2_ref[0] = m2.astype(jnp.bfloat16)


def _bit_of(bytes_f32, p):
    t = jnp.floor(bytes_f32 * _pow2(-p))
    return t - 2.0 * jnp.floor(t * 0.5)


def _output_body(z_i_ref, z_j_ref, m1_ref, m2_ref, o_ref):
    zi = z_i_ref[0].astype(jnp.bfloat16)
    zj = z_j_ref[0].astype(jnp.bfloat16)
    s = jax.lax.dot_general(
        zi, zj, (((1,), (1,)), ((), ())),
        preferred_element_type=jnp.float32)
    a = jax.nn.sigmoid(s)

    cols = jax.lax.broadcasted_iota(jnp.int32, (P2_I, P2_J), 1)

    ei = jax.lax.broadcasted_iota(jnp.int32, (P2_J // 8, P2_J), 0)
    ej = jax.lax.broadcasted_iota(jnp.int32, (P2_J // 8, P2_J), 1)
    e1 = (ej >> 3 == ei).astype(jnp.bfloat16)
    b1 = jax.lax.dot_general(
        m1_ref[0].astype(jnp.bfloat16), e1, (((1,), (0,)), ((), ())),
        preferred_element_type=jnp.float32)
    mask = _bit_of(b1, cols & 7)
    b2 = jax.lax.dot_general(
        m2_ref[0].astype(jnp.bfloat16), e1, (((0,), (0,)), ((), ())),
        preferred_element_type=jnp.float32)
    mask_t = _bit_of(b2, cols & 7)

    o_ref[0] = jnp.where(mask + mask_t > 0.0, a, 0.0)


def _build(z, interpret=False):
    b, n, d = z.shape
    nrb = n // ROW_BLK

    m1, m2 = pl.pallas_call(
        _sparsify_body,
        grid=(b, nrb),
        in_specs=[
            pl.BlockSpec((1, n, d), lambda bi, i: (bi, 0, 0)),
            pl.BlockSpec((1, ROW_BLK, d), lambda bi, i: (bi, i, 0)),
        ],
        out_specs=[
            pl.BlockSpec((1, ROW_BLK, n // 8), lambda bi, i: (bi, i, 0)),
            pl.BlockSpec((1, ROW_BLK // 8, n), lambda bi, i: (bi, i, 0)),
        ],
        out_shape=[
            jax.ShapeDtypeStruct((b, n, n // 8), jnp.bfloat16),
            jax.ShapeDtypeStruct((b, n // 8, n), jnp.bfloat16),
        ],
        scratch_shapes=[
            pltpu.VMEM((n, ROW_BLK), jnp.float32),
        ],
        interpret=interpret,
    )(z, z)

    out = pl.pallas_call(
        _output_body,
        grid=(b, n // P2_I, n // P2_J),
        in_specs=[
            pl.BlockSpec((1, P2_I, d), lambda bi, i, j: (bi, i, 0)),
            pl.BlockSpec((1, P2_J, d), lambda bi, i, j: (bi, j, 0)),
            pl.BlockSpec((1, P2_I, P2_J // 8), lambda bi, i, j: (bi, i, j)),
            pl.BlockSpec((1, P2_J // 8, P2_I), lambda bi, i, j: (bi, j, i)),
        ],
        out_specs=pl.BlockSpec((1, P2_I, P2_J), lambda bi, i, j: (bi, i, j)),
        out_shape=jax.ShapeDtypeStruct((b, n, n), jnp.float32),
        interpret=interpret,
    )(z, z, m1, m2)
    return out


@jax.jit
def kernel(z):
    return _build(z)


# speedup vs baseline: 2.1457x; 1.0148x over previous
import jax
import jax.numpy as jnp
from jax.experimental import pallas as pl
from jax.experimental.pallas import tpu as pltpu

TOPK_K = 32
ROW_BLK = 1024
TIE_CHUNK = 512
P2_I = 512
P2_J = 2048


def _pow2(e):
    return jax.lax.bitcast_convert_type((e + 127) << 23, jnp.float32)


def _sparsify_body(z_all_ref, z_blk_ref, m1_ref, m2_ref, a_scr):
    i = pl.program_id(1)
    za = z_all_ref[0].astype(jnp.bfloat16)
    zb = z_blk_ref[0].astype(jnp.bfloat16)
    w = jax.lax.dot_general(
        za, zb, (((1,), (1,)), ((), ())),
        preferred_element_type=jnp.float32)
    a = jax.nn.sigmoid(w)
    n = a.shape[0]
    ci = jax.lax.broadcasted_iota(jnp.int32, a.shape, 0)
    ri = jax.lax.broadcasted_iota(jnp.int32, a.shape, 1) + i * ROW_BLK
    a = jnp.where(ci == ri, 0.0, a)
    a_scr[...] = a

    x = a
    for half in (1024, 512, 256, 128, 64):
        x = jnp.maximum(x[:half], x[half:])
    cmax = x
    hi0 = jax.lax.bitcast_convert_type(
        jnp.max(cmax, axis=0, keepdims=True), jnp.int32)

    def seed_search(_, lohi):
        lo, hi = lohi
        mid = (lo + hi + 1) >> 1
        thr = jax.lax.bitcast_convert_type(mid, jnp.float32)
        cnt = jnp.sum((cmax >= thr).astype(jnp.int32), axis=0,
                      keepdims=True)
        ge = cnt >= TOPK_K
        return jnp.where(ge, mid, lo), jnp.where(ge, hi, mid - 1)

    lo0, _ = jax.lax.fori_loop(
        0, 31, seed_search, (jnp.zeros((1, ROW_BLK), jnp.int32), hi0))

    def not_done(lohi):
        lo, hi = lohi
        return jnp.any(lo < hi)

    def search(lohi):
        lo, hi = lohi
        mid = (lo + hi + 1) >> 1
        thr = jax.lax.bitcast_convert_type(mid, jnp.float32)
        cnt = jnp.sum((a_scr[...] >= thr).astype(jnp.int32), axis=0,
                      keepdims=True)
        ge = cnt >= TOPK_K
        return jnp.where(ge, mid, lo), jnp.where(ge, hi, mid - 1)

    lo, _ = jax.lax.while_loop(not_done, search, (lo0, hi0))
    v32 = jax.lax.bitcast_convert_type(lo, jnp.float32)

    av = a_scr[...]
    gt = av > v32
    eq = av == v32
    cnt_gt = jnp.sum(gt.astype(jnp.int32), axis=0, keepdims=True)
    r = (TOPK_K - cnt_gt).astype(jnp.float32)

    t0 = jax.lax.broadcasted_iota(jnp.int32, (TIE_CHUNK, TIE_CHUNK), 0)
    t1 = jax.lax.broadcasted_iota(jnp.int32, (TIE_CHUNK, TIE_CHUNK), 1)
    tril = (t1 < t0).astype(jnp.bfloat16)
    eqb = eq.astype(jnp.bfloat16)
    keeps = []
    carry = jnp.zeros((1, ROW_BLK), jnp.float32)
    for c in range(n // TIE_CHUNK):
        eqc = eqb[c * TIE_CHUNK:(c + 1) * TIE_CHUNK, :]
        excl = jax.lax.dot_general(
            tril, eqc, (((1,), (0,)), ((), ())),
            preferred_element_type=jnp.float32) + carry
        sl = slice(c * TIE_CHUNK, (c + 1) * TIE_CHUNK)
        keeps.append(jnp.logical_or(gt[sl], jnp.logical_and(eq[sl],
                                                            excl < r)))
        carry = carry + jnp.sum(eqc.astype(jnp.float32), axis=0,
                                keepdims=True)
    keep = jnp.concatenate(keeps, axis=0).astype(jnp.bfloat16)

    gi = jax.lax.broadcasted_iota(jnp.int32, (n, n // 8), 0)
    gj = jax.lax.broadcasted_iota(jnp.int32, (n, n // 8), 1)
    bsub = jnp.where(gi >> 3 == gj, _pow2(gi & 7), 0.0).astype(jnp.bfloat16)
    m1 = jax.lax.dot_general(
        keep, bsub, (((0,), (0,)), ((), ())),
        preferred_element_type=jnp.float32)
    m1_ref[0] = m1.astype(jnp.bfloat16)
    hi2 = jax.lax.broadcasted_iota(jnp.int32, (ROW_BLK, ROW_BLK // 8), 0)
    hj2 = jax.lax.broadcasted_iota(jnp.int32, (ROW_BLK, ROW_BLK // 8), 1)
    b2 = jnp.where(hi2 >> 3 == hj2, _pow2(hi2 & 7), 0.0).astype(jnp.bfloat16)
    m2 = jax.lax.dot_general(
        b2, keep, (((0,), (1,)), ((), ())),
        preferred_element_type=jnp.float32)
    m2_ref[0] = m2.astype(jnp.bfloat16)


def _bit_of(bytes_f32, p):
    t = jnp.floor(bytes_f32 * _pow2(-p))
    return t - 2.0 * jnp.floor(t * 0.5)


def _output_body(z_i_ref, z_j_ref, m1_ref, m2_ref, o_ref):
    zi = z_i_ref[0].astype(jnp.bfloat16)
    zj = z_j_ref[0].astype(jnp.bfloat16)
    s = jax.lax.dot_general(
        zi, zj, (((1,), (1,)), ((), ())),
        preferred_element_type=jnp.float32)
    a = jax.nn.sigmoid(s)

    cols = jax.lax.broadcasted_iota(jnp.int32, (P2_I, P2_J), 1)

    ei = jax.lax.broadcasted_iota(jnp.int32, (P2_J // 8, P2_J), 0)
    ej = jax.lax.broadcasted_iota(jnp.int32, (P2_J // 8, P2_J), 1)
    e1 = (ej >> 3 == ei).astype(jnp.bfloat16)
    b1 = jax.lax.dot_general(
        m1_ref[0].astype(jnp.bfloat16), e1, (((1,), (0,)), ((), ())),
        preferred_element_type=jnp.float32)
    mask = _bit_of(b1, cols & 7)
    b2 = jax.lax.dot_general(
        m2_ref[0].astype(jnp.bfloat16), e1, (((0,), (0,)), ((), ())),
        preferred_element_type=jnp.float32)
    mask_t = _bit_of(b2, cols & 7)

    o_ref[0] = jnp.where(mask + mask_t > 0.0, a, 0.0)


def _build(z, interpret=False):
    b, n, d = z.shape
    nrb = n // ROW_BLK

    m1, m2 = pl.pallas_call(
        _sparsify_body,
        grid=(b, nrb),
        in_specs=[
            pl.BlockSpec((1, n, d), lambda bi, i: (bi, 0, 0)),
            pl.BlockSpec((1, ROW_BLK, d), lambda bi, i: (bi, i, 0)),
        ],
        out_specs=[
            pl.BlockSpec((1, ROW_BLK, n // 8), lambda bi, i: (bi, i, 0)),
            pl.BlockSpec((1, ROW_BLK // 8, n), lambda bi, i: (bi, i, 0)),
        ],
        out_shape=[
            jax.ShapeDtypeStruct((b, n, n // 8), jnp.bfloat16),
            jax.ShapeDtypeStruct((b, n // 8, n), jnp.bfloat16),
        ],
        scratch_shapes=[
            pltpu.VMEM((n, ROW_BLK), jnp.float32),
        ],
        interpret=interpret,
    )(z, z)

    out = pl.pallas_call(
        _output_body,
        grid=(b, n // P2_I, n // P2_J),
        in_specs=[
            pl.BlockSpec((1, P2_I, d), lambda bi, i, j: (bi, i, 0)),
            pl.BlockSpec((1, P2_J, d), lambda bi, i, j: (bi, j, 0)),
            pl.BlockSpec((1, P2_I, P2_J // 8), lambda bi, i, j: (bi, i, j)),
            pl.BlockSpec((1, P2_J // 8, P2_I), lambda bi, i, j: (bi, j, i)),
        ],
        out_specs=pl.BlockSpec((1, P2_I, P2_J), lambda bi, i, j: (bi, i, j)),
        out_shape=jax.ShapeDtypeStruct((b, n, n), jnp.float32),
        interpret=interpret,
    )(z, z, m1, m2)
    return out


@jax.jit
def kernel(z):
    return _build(z)
